# baseline (speedup 1.0000x reference)
import functools
import jax
import jax.numpy as jnp
from jax import lax
from jax.experimental import pallas as pl
from jax.experimental.pallas import tpu as pltpu
from jax.experimental.pallas import tpu_sc as plsc

_H = 224
_W = 224
_D = 224
_S = 56
_N = _H * _W * _D
_NC = 2
_NS = 16
_NW = _NC * _NS
_ROWS = _S * _H
_RPW = _ROWS // _NW
_HALF = _RPW // 2
_G = _W // 16
_FMAX = 223.0

_SCRATCH = [
    pltpu.VMEM((_S * 12,), jnp.float32),
    pltpu.VMEM((2, 2, 112), jnp.int32),
    pltpu.VMEM((2, _W, 8), jnp.float32),
    pltpu.VMEM((2, 3, _W), jnp.float32),
    pltpu.VMEM((2, _W), jnp.float32),
    pltpu.SemaphoreType.DMA,
    pltpu.SemaphoreType.DMA,
    pltpu.SemaphoreType.DMA,
    pltpu.SemaphoreType.DMA,
]


def _bf16_round(x):
    u = plsc.bitcast(x, jnp.uint32)
    r = (u + jnp.uint32(0x7FFF) + ((u >> jnp.uint32(16)) & jnp.uint32(1))) \
        & jnp.uint32(0xFFFF0000)
    return plsc.bitcast(r, jnp.float32)


def _v2s_body(tab_hbm, trf_hbm, out_hbm,
              trf_v, idx_v, val_v, frc_v, row_v, g0, g1, o0, o1):
    gsem = (g0, g1)
    osem = (o0, o1)
    wid = lax.axis_index("sub") * _NC + lax.axis_index("core")
    wbase = wid * _RPW
    pltpu.sync_copy(trf_hbm, trf_v)

    lane_i = lax.broadcasted_iota(jnp.int32, (16,), 0)
    lane = lane_i.astype(jnp.float32)

    def compute_idx(r, b):
        s = r // _H
        i = r % _H
        base = s * 12
        c = [plsc.load_gather(trf_v, [jnp.full((16,), base + k, jnp.int32)])
             for k in range(12)]
        one = jnp.float32(1.0)
        a00 = _bf16_round(c[0] + one)
        a11 = _bf16_round(c[5] + one)
        a22 = _bf16_round(c[10] + one)
        c = [_bf16_round(ck) for ck in c]
        fi = jnp.full((16,), i, jnp.int32).astype(jnp.float32)
        fk = jnp.full((16,), 4 * s, jnp.int32).astype(jnp.float32)
        cx = fi * a00 + fk * c[2] + c[3]
        cy = fi * c[4] + fk * c[6] + c[7]
        cz = fi * c[8] + fk * a22 + c[11]
        for g in range(_G):
            jv = lane + jnp.float32(g * 16)
            x = jnp.clip(cx + jv * c[1], 0.0, _FMAX)
            y = jnp.clip(cy + jv * a11, 0.0, _FMAX)
            z = jnp.clip(cz + jv * c[9], 0.0, _FMAX)
            x0 = x.astype(jnp.int32)
            y0 = y.astype(jnp.int32)
            z0 = z.astype(jnp.int32)
            frc_v[b, 0, pl.ds(g * 16, 16)] = x - x0.astype(jnp.float32)
            frc_v[b, 1, pl.ds(g * 16, 16)] = y - y0.astype(jnp.float32)
            frc_v[b, 2, pl.ds(g * 16, 16)] = z - z0.astype(jnp.float32)
            idx_v[b, g // 7, pl.ds((g % 7) * 16, 16)] = \
                x0 * (_W * _D) + y0 * _D + z0

    def fire(b):
        for h in range(2):
            pltpu.make_async_copy(tab_hbm.at[idx_v.at[b, h]],
                                  val_v.at[b, pl.ds(h * 112, 112)],
                                  gsem[b]).start()

    def drain(b):
        for h in range(2):
            pltpu.make_async_copy(tab_hbm.at[idx_v.at[b, h]],
                                  val_v.at[b, pl.ds(h * 112, 112)],
                                  gsem[b]).wait()

    def combine(r, b):
        for g in range(_G):
            pv = lane_i + (g * 16)
            vs = [plsc.load_gather(val_v.at[b],
                                   [pv, jnp.full((16,), cc, jnp.int32)])
                  for cc in range(8)]
            dx = frc_v[b, 0, pl.ds(g * 16, 16)]
            dy = frc_v[b, 1, pl.ds(g * 16, 16)]
            dz = frc_v[b, 2, pl.ds(g * 16, 16)]
            v00 = vs[0] + dz * (vs[1] - vs[0])
            v01 = vs[2] + dz * (vs[3] - vs[2])
            v10 = vs[4] + dz * (vs[5] - vs[4])
            v11 = vs[6] + dz * (vs[7] - vs[6])
            v0 = v00 + dy * (v01 - v00)
            v1 = v10 + dy * (v11 - v10)
            row_v[b, pl.ds(g * 16, 16)] = v0 + dx * (v1 - v0)

    def fire_out(r, b):
        s = r // _H
        i = r % _H
        pltpu.make_async_copy(row_v.at[b], out_hbm.at[s, i], osem[b]).start()

    def wait_out(r, b):
        s = r // _H
        i = r % _H
        pltpu.make_async_copy(row_v.at[b], out_hbm.at[s, i], osem[b]).wait()

    compute_idx(wbase, 0)
    fire(0)

    def body(u, carry):
        r0 = wbase + 2 * u
        compute_idx(r0 + 1, 1)
        fire(1)
        drain(0)
        pl.when(u > 0)(lambda: wait_out(r0 - 2, 0))
        combine(r0, 0)
        fire_out(r0, 0)

        def prefetch():
            compute_idx(r0 + 2, 0)
            fire(0)
        pl.when(u < _HALF - 1)(prefetch)
        drain(1)
        pl.when(u > 0)(lambda: wait_out(r0 - 1, 1))
        combine(r0 + 1, 1)
        fire_out(r0 + 1, 1)
        return carry

    lax.fori_loop(0, _HALF, body, 0)
    wait_out(wbase + _RPW - 2, 0)
    wait_out(wbase + _RPW - 1, 1)


_v2s_kernel = functools.partial(
    pl.kernel,
    out_type=jax.ShapeDtypeStruct((_S, _H, _W), jnp.float32),
    mesh=plsc.VectorSubcoreMesh(core_axis_name="core", subcore_axis_name="sub",
                                num_cores=_NC, num_subcores=_NS),
    scratch_types=_SCRATCH,
    compiler_params=pltpu.CompilerParams(needs_layout_passes=False,
                                         use_tc_tiling_on_sc=False),
)(_v2s_body)


@jax.jit
def kernel(vol, trf):
    v = vol[0, ..., 0]
    vz = jnp.concatenate([v[:, :, 1:], v[:, :, _D - 1:]], axis=2)
    pz = jnp.stack([v, vz], axis=-1)
    py = jnp.concatenate([pz[:, 1:], pz[:, _W - 1:]], axis=1)
    qu = jnp.concatenate([pz, py], axis=-1)
    px = jnp.concatenate([qu[1:], qu[_H - 1:]], axis=0)
    oc = jnp.concatenate([qu, px], axis=-1)
    tab = oc.reshape(_N, 8)
    t = trf.reshape(_S * 12)
    out3 = _v2s_kernel(tab, t)
    return jnp.transpose(out3, (1, 2, 0))[None, ..., None]


# speedup vs baseline: 8.5549x; 8.5549x over previous
import functools
import jax
import jax.numpy as jnp
from jax import lax
from jax.experimental import pallas as pl
from jax.experimental.pallas import tpu as pltpu
from jax.experimental.pallas import tpu_sc as plsc

_H = 224
_W = 224
_D = 224
_S = 56
_N = _H * _W * _D
_NC = 2
_NS = 16
_NW = _NC * _NS
_ROWS = _S * _H
_RPW = _ROWS // _NW
_G = _W // 16
_FMAX = 223.0
_IMAX = 223

_SCRATCH = [
    pltpu.VMEM((_S * 12,), jnp.float32),
    pltpu.VMEM((_G, 128), jnp.int32),
    pltpu.VMEM((_G, 128), jnp.float32),
    pltpu.VMEM((_W,), jnp.float32),
    pltpu.VMEM((_W,), jnp.float32),
    pltpu.VMEM((_W,), jnp.float32),
    pltpu.VMEM((_W,), jnp.float32),
    pltpu.SemaphoreType.DMA,
    pltpu.SemaphoreType.DMA,
]


def _bf16_round(x):
    u = plsc.bitcast(x, jnp.uint32)
    r = (u + jnp.uint32(0x7FFF) + ((u >> jnp.uint32(16)) & jnp.uint32(1))) \
        & jnp.uint32(0xFFFF0000)
    return plsc.bitcast(r, jnp.float32)


def _v2s_body(vol_hbm, trf_hbm, out_hbm,
              trf_v, idx_v, val_v, dx_v, dy_v, dz_v, row_v, gsem, osem):
    wid = lax.axis_index("sub") * _NC + lax.axis_index("core")
    pltpu.sync_copy(trf_hbm, trf_v)

    def row_body(t, carry):
        r = wid * _RPW + t
        s = r // _H
        i = r % _H

        base = s * 12
        c = [plsc.load_gather(trf_v, [jnp.full((16,), base + k, jnp.int32)])
             for k in range(12)]
        one = jnp.float32(1.0)
        a00 = _bf16_round(c[0] + one)
        a11 = _bf16_round(c[5] + one)
        a22 = _bf16_round(c[10] + one)
        c = [_bf16_round(ck) for ck in c]
        fi = jnp.full((16,), i, jnp.int32).astype(jnp.float32)
        fk = jnp.full((16,), 4 * s, jnp.int32).astype(jnp.float32)
        cx = fi * a00 + fk * c[2] + c[3]
        cy = fi * c[4] + fk * c[6] + c[7]
        cz = fi * c[8] + fk * a22 + c[11]
        ax = c[1]
        ay = a11
        az = c[9]

        lane = lax.broadcasted_iota(jnp.int32, (16,), 0).astype(jnp.float32)
        for g in range(_G):
            jv = lane + jnp.float32(g * 16)
            x = jnp.clip(cx + jv * ax, 0.0, _FMAX)
            y = jnp.clip(cy + jv * ay, 0.0, _FMAX)
            z = jnp.clip(cz + jv * az, 0.0, _FMAX)
            x0 = x.astype(jnp.int32)
            y0 = y.astype(jnp.int32)
            z0 = z.astype(jnp.int32)
            dx_v[pl.ds(g * 16, 16)] = x - x0.astype(jnp.float32)
            dy_v[pl.ds(g * 16, 16)] = y - y0.astype(jnp.float32)
            dz_v[pl.ds(g * 16, 16)] = z - z0.astype(jnp.float32)
            x1 = jnp.minimum(x0 + 1, _IMAX)
            y1 = jnp.minimum(y0 + 1, _IMAX)
            z1 = jnp.minimum(z0 + 1, _IMAX)
            xb0 = x0 * (_W * _D)
            xb1 = x1 * (_W * _D)
            yb0 = y0 * _D
            yb1 = y1 * _D
            b00 = xb0 + yb0
            b01 = xb0 + yb1
            b10 = xb1 + yb0
            b11 = xb1 + yb1
            idx_v[g, pl.ds(0, 16)] = b00 + z0
            idx_v[g, pl.ds(16, 16)] = b00 + z1
            idx_v[g, pl.ds(32, 16)] = b01 + z0
            idx_v[g, pl.ds(48, 16)] = b01 + z1
            idx_v[g, pl.ds(64, 16)] = b10 + z0
            idx_v[g, pl.ds(80, 16)] = b10 + z1
            idx_v[g, pl.ds(96, 16)] = b11 + z0
            idx_v[g, pl.ds(112, 16)] = b11 + z1

        copies = [pltpu.make_async_copy(vol_hbm.at[idx_v.at[g]],
                                        val_v.at[g], gsem)
                  for g in range(_G)]
        for cp in copies:
            cp.start()
        for cp in copies:
            cp.wait()

        for g in range(_G):
            v000 = val_v[g, pl.ds(0, 16)]
            v001 = val_v[g, pl.ds(16, 16)]
            v010 = val_v[g, pl.ds(32, 16)]
            v011 = val_v[g, pl.ds(48, 16)]
            v100 = val_v[g, pl.ds(64, 16)]
            v101 = val_v[g, pl.ds(80, 16)]
            v110 = val_v[g, pl.ds(96, 16)]
            v111 = val_v[g, pl.ds(112, 16)]
            dx = dx_v[pl.ds(g * 16, 16)]
            dy = dy_v[pl.ds(g * 16, 16)]
            dz = dz_v[pl.ds(g * 16, 16)]
            v00 = v000 + dz * (v001 - v000)
            v01 = v010 + dz * (v011 - v010)
            v10 = v100 + dz * (v101 - v100)
            v11 = v110 + dz * (v111 - v110)
            v0 = v00 + dy * (v01 - v00)
            v1 = v10 + dy * (v11 - v10)
            row_v[pl.ds(g * 16, 16)] = v0 + dx * (v1 - v0)

        pltpu.async_copy(row_v, out_hbm.at[s, i], osem).wait()
        return carry

    lax.fori_loop(0, _RPW, row_body, 0)


_v2s_kernel = functools.partial(
    pl.kernel,
    out_type=jax.ShapeDtypeStruct((_S, _H, _W), jnp.float32),
    mesh=plsc.VectorSubcoreMesh(core_axis_name="core", subcore_axis_name="sub",
                                num_cores=_NC, num_subcores=_NS),
    scratch_types=_SCRATCH,
    compiler_params=pltpu.CompilerParams(needs_layout_passes=False),
)(_v2s_body)


@jax.jit
def kernel(vol, trf):
    v = vol.reshape(_N)
    t = trf.reshape(_S * 12)
    out3 = _v2s_kernel(v, t)
    return jnp.transpose(out3, (1, 2, 0))[None, ..., None]


# speedup vs baseline: 10.4135x; 1.2173x over previous
import functools
import jax
import jax.numpy as jnp
from jax import lax
from jax.experimental import pallas as pl
from jax.experimental.pallas import tpu as pltpu
from jax.experimental.pallas import tpu_sc as plsc

_H = 224
_W = 224
_D = 224
_S = 56
_N = _H * _W * _D
_NC = 2
_NS = 16
_NW = _NC * _NS
_ROWS = _S * _H
_RPW = _ROWS // _NW
_HALF = _RPW // 2
_G = _W // 16
_E = _W * 8
_FMAX = 223.0
_IMAX = 223

_SCRATCH = [
    pltpu.VMEM((_S * 12,), jnp.float32),
    pltpu.VMEM((_E,), jnp.int32),
    pltpu.VMEM((_E,), jnp.int32),
    pltpu.VMEM((_E,), jnp.float32),
    pltpu.VMEM((_E,), jnp.float32),
    pltpu.VMEM((3, _W), jnp.float32),
    pltpu.VMEM((3, _W), jnp.float32),
    pltpu.VMEM((_W,), jnp.float32),
    pltpu.VMEM((_W,), jnp.float32),
    pltpu.SemaphoreType.DMA,
    pltpu.SemaphoreType.DMA,
    pltpu.SemaphoreType.DMA,
    pltpu.SemaphoreType.DMA,
]


def _bf16_round(x):
    u = plsc.bitcast(x, jnp.uint32)
    r = (u + jnp.uint32(0x7FFF) + ((u >> jnp.uint32(16)) & jnp.uint32(1))) \
        & jnp.uint32(0xFFFF0000)
    return plsc.bitcast(r, jnp.float32)


def _v2s_body(vol_hbm, trf_hbm, out_hbm,
              trf_v, idx0, idx1, val0, val1, frc0, frc1, row0, row1,
              g0, g1, o0, o1):
    idxs = (idx0, idx1)
    vals = (val0, val1)
    frcs = (frc0, frc1)
    rows = (row0, row1)
    gsem = (g0, g1)
    osem = (o0, o1)
    wid = lax.axis_index("sub") * _NC + lax.axis_index("core")
    wbase = wid * _RPW
    pltpu.sync_copy(trf_hbm, trf_v)

    lane_i = lax.broadcasted_iota(jnp.int32, (16,), 0)
    lane = lane_i.astype(jnp.float32)

    def compute_idx(r, b):
        idx_v = idxs[b]
        frc_v = frcs[b]
        s = r // _H
        i = r % _H
        base = s * 12
        c = [plsc.load_gather(trf_v, [jnp.full((16,), base + k, jnp.int32)])
             for k in range(12)]
        one = jnp.float32(1.0)
        a00 = _bf16_round(c[0] + one)
        a11 = _bf16_round(c[5] + one)
        a22 = _bf16_round(c[10] + one)
        c = [_bf16_round(ck) for ck in c]
        fi = jnp.full((16,), i, jnp.int32).astype(jnp.float32)
        fk = jnp.full((16,), 4 * s, jnp.int32).astype(jnp.float32)
        cx = fi * a00 + fk * c[2] + c[3]
        cy = fi * c[4] + fk * c[6] + c[7]
        cz = fi * c[8] + fk * a22 + c[11]
        for g in range(_G):
            jv = lane + jnp.float32(g * 16)
            x = jnp.clip(cx + jv * c[1], 0.0, _FMAX)
            y = jnp.clip(cy + jv * a11, 0.0, _FMAX)
            z = jnp.clip(cz + jv * c[9], 0.0, _FMAX)
            x0 = x.astype(jnp.int32)
            y0 = y.astype(jnp.int32)
            z0 = z.astype(jnp.int32)
            frc_v[0, pl.ds(g * 16, 16)] = x - x0.astype(jnp.float32)
            frc_v[1, pl.ds(g * 16, 16)] = y - y0.astype(jnp.float32)
            frc_v[2, pl.ds(g * 16, 16)] = z - z0.astype(jnp.float32)
            x1 = jnp.minimum(x0 + 1, _IMAX)
            y1 = jnp.minimum(y0 + 1, _IMAX)
            z1 = jnp.minimum(z0 + 1, _IMAX)
            xb0 = x0 * (_W * _D)
            xb1 = x1 * (_W * _D)
            yb0 = y0 * _D
            yb1 = y1 * _D
            b00 = xb0 + yb0
            b01 = xb0 + yb1
            b10 = xb1 + yb0
            b11 = xb1 + yb1
            e = g * 128
            idx_v[pl.ds(e, 16)] = b00 + z0
            idx_v[pl.ds(e + 16, 16)] = b00 + z1
            idx_v[pl.ds(e + 32, 16)] = b01 + z0
            idx_v[pl.ds(e + 48, 16)] = b01 + z1
            idx_v[pl.ds(e + 64, 16)] = b10 + z0
            idx_v[pl.ds(e + 80, 16)] = b10 + z1
            idx_v[pl.ds(e + 96, 16)] = b11 + z0
            idx_v[pl.ds(e + 112, 16)] = b11 + z1

    def fire(b):
        pltpu.make_async_copy(vol_hbm.at[idxs[b]],
                              vals[b], gsem[b]).start()

    def drain(b):
        pltpu.make_async_copy(vol_hbm.at[idxs[b]],
                              vals[b], gsem[b]).wait()

    def combine(b):
        val_v = vals[b]
        frc_v = frcs[b]
        row_v = rows[b]
        for g in range(_G):
            e = g * 128
            v000 = val_v[pl.ds(e, 16)]
            v001 = val_v[pl.ds(e + 16, 16)]
            v010 = val_v[pl.ds(e + 32, 16)]
            v011 = val_v[pl.ds(e + 48, 16)]
            v100 = val_v[pl.ds(e + 64, 16)]
            v101 = val_v[pl.ds(e + 80, 16)]
            v110 = val_v[pl.ds(e + 96, 16)]
            v111 = val_v[pl.ds(e + 112, 16)]
            dx = frc_v[0, pl.ds(g * 16, 16)]
            dy = frc_v[1, pl.ds(g * 16, 16)]
            dz = frc_v[2, pl.ds(g * 16, 16)]
            v00 = v000 + dz * (v001 - v000)
            v01 = v010 + dz * (v011 - v010)
            v10 = v100 + dz * (v101 - v100)
            v11 = v110 + dz * (v111 - v110)
            v0 = v00 + dy * (v01 - v00)
            v1 = v10 + dy * (v11 - v10)
            row_v[pl.ds(g * 16, 16)] = v0 + dx * (v1 - v0)

    def fire_out(r, b):
        s = r // _H
        i = r % _H
        pltpu.make_async_copy(rows[b], out_hbm.at[s, i], osem[b]).start()

    def wait_out(r, b):
        s = r // _H
        i = r % _H
        pltpu.make_async_copy(rows[b], out_hbm.at[s, i], osem[b]).wait()

    compute_idx(wbase, 0)
    fire(0)

    def body(u, carry):
        r0 = wbase + 2 * u
        compute_idx(r0 + 1, 1)
        fire(1)
        drain(0)
        pl.when(u > 0)(lambda: wait_out(r0 - 2, 0))
        combine(0)
        fire_out(r0, 0)

        def prefetch():
            compute_idx(r0 + 2, 0)
            fire(0)
        pl.when(u < _HALF - 1)(prefetch)
        drain(1)
        pl.when(u > 0)(lambda: wait_out(r0 - 1, 1))
        combine(1)
        fire_out(r0 + 1, 1)
        return carry

    lax.fori_loop(0, _HALF, body, 0)
    wait_out(wbase + _RPW - 2, 0)
    wait_out(wbase + _RPW - 1, 1)


_v2s_kernel = functools.partial(
    pl.kernel,
    out_type=jax.ShapeDtypeStruct((_S, _H, _W), jnp.float32),
    mesh=plsc.VectorSubcoreMesh(core_axis_name="core", subcore_axis_name="sub",
                                num_cores=_NC, num_subcores=_NS),
    scratch_types=_SCRATCH,
    compiler_params=pltpu.CompilerParams(needs_layout_passes=False),
)(_v2s_body)


@jax.jit
def kernel(vol, trf):
    v = vol.reshape(_N)
    t = trf.reshape(_S * 12)
    out3 = _v2s_kernel(v, t)
    return jnp.transpose(out3, (1, 2, 0))[None, ..., None]


# speedup vs baseline: 10.9054x; 1.0472x over previous
import functools
import jax
import jax.numpy as jnp
from jax import lax
from jax.experimental import pallas as pl
from jax.experimental.pallas import tpu as pltpu
from jax.experimental.pallas import tpu_sc as plsc

_H = 224
_W = 224
_D = 224
_S = 56
_N = _H * _W * _D
_NC = 2
_NS = 16
_NW = _NC * _NS
_ROWS = _S * _H
_RPW = _ROWS // _NW
_HALF = _RPW // 2
_G = _W // 16
_E = _W * 8
_FMAX = 223.0
_IMAX = 223

_SCRATCH = [
    pltpu.VMEM((_S * 12,), jnp.float32),
    pltpu.VMEM((_E,), jnp.int32),
    pltpu.VMEM((_E,), jnp.int32),
    pltpu.VMEM((_E,), jnp.float32),
    pltpu.VMEM((_E,), jnp.float32),
    pltpu.VMEM((3, _W), jnp.float32),
    pltpu.VMEM((3, _W), jnp.float32),
    pltpu.VMEM((_W,), jnp.float32),
    pltpu.VMEM((_W,), jnp.float32),
    pltpu.SemaphoreType.DMA,
    pltpu.SemaphoreType.DMA,
    pltpu.SemaphoreType.DMA,
    pltpu.SemaphoreType.DMA,
]


def _bf16_round(x):
    u = plsc.bitcast(x, jnp.uint32)
    r = (u + jnp.uint32(0x7FFF) + ((u >> jnp.uint32(16)) & jnp.uint32(1))) \
        & jnp.uint32(0xFFFF0000)
    return plsc.bitcast(r, jnp.float32)


def _v2s_body(vol_hbm, trf_hbm, out_hbm,
              trf_v, idx0, idx1, val0, val1, frc0, frc1, row0, row1,
              g0, g1, o0, o1):
    idxs = (idx0, idx1)
    vals = (val0, val1)
    frcs = (frc0, frc1)
    rows = (row0, row1)
    gsem = (g0, g1)
    osem = (o0, o1)
    wid = lax.axis_index("sub") * _NC + lax.axis_index("core")
    pltpu.sync_copy(trf_hbm, trf_v)

    lane_i = lax.broadcasted_iota(jnp.int32, (16,), 0)
    lane = lane_i.astype(jnp.float32)

    def compute_idx(r, b):
        idx_v = idxs[b]
        frc_v = frcs[b]
        s = r // _H
        i = r % _H
        base = s * 12
        c = [plsc.load_gather(trf_v, [jnp.full((16,), base + k, jnp.int32)])
             for k in range(12)]
        one = jnp.float32(1.0)
        a00 = _bf16_round(c[0] + one)
        a11 = _bf16_round(c[5] + one)
        a22 = _bf16_round(c[10] + one)
        c = [_bf16_round(ck) for ck in c]
        fi = jnp.full((16,), i, jnp.int32).astype(jnp.float32)
        fk = jnp.full((16,), 4 * s, jnp.int32).astype(jnp.float32)
        cx = fi * a00 + fk * c[2] + c[3]
        cy = fi * c[4] + fk * c[6] + c[7]
        cz = fi * c[8] + fk * a22 + c[11]
        for g in range(_G):
            jv = lane + jnp.float32(g * 16)
            x = jnp.clip(cx + jv * c[1], 0.0, _FMAX)
            y = jnp.clip(cy + jv * a11, 0.0, _FMAX)
            z = jnp.clip(cz + jv * c[9], 0.0, _FMAX)
            x0 = x.astype(jnp.int32)
            y0 = y.astype(jnp.int32)
            z0 = z.astype(jnp.int32)
            frc_v[0, pl.ds(g * 16, 16)] = x - x0.astype(jnp.float32)
            frc_v[1, pl.ds(g * 16, 16)] = y - y0.astype(jnp.float32)
            frc_v[2, pl.ds(g * 16, 16)] = z - z0.astype(jnp.float32)
            x1 = jnp.minimum(x0 + 1, _IMAX)
            y1 = jnp.minimum(y0 + 1, _IMAX)
            z1 = jnp.minimum(z0 + 1, _IMAX)
            xb0 = x0 * (_W * _D)
            xb1 = x1 * (_W * _D)
            yb0 = y0 * _D
            yb1 = y1 * _D
            b00 = xb0 + yb0
            b01 = xb0 + yb1
            b10 = xb1 + yb0
            b11 = xb1 + yb1
            e = g * 128
            idx_v[pl.ds(e, 16)] = b00 + z0
            idx_v[pl.ds(e + 16, 16)] = b00 + z1
            idx_v[pl.ds(e + 32, 16)] = b01 + z0
            idx_v[pl.ds(e + 48, 16)] = b01 + z1
            idx_v[pl.ds(e + 64, 16)] = b10 + z0
            idx_v[pl.ds(e + 80, 16)] = b10 + z1
            idx_v[pl.ds(e + 96, 16)] = b11 + z0
            idx_v[pl.ds(e + 112, 16)] = b11 + z1

    def fire(b):
        pltpu.make_async_copy(vol_hbm.at[idxs[b]],
                              vals[b], gsem[b]).start()

    def drain(b):
        pltpu.make_async_copy(vol_hbm.at[idxs[b]],
                              vals[b], gsem[b]).wait()

    def combine(b):
        val_v = vals[b]
        frc_v = frcs[b]
        row_v = rows[b]
        for g in range(_G):
            e = g * 128
            v000 = val_v[pl.ds(e, 16)]
            v001 = val_v[pl.ds(e + 16, 16)]
            v010 = val_v[pl.ds(e + 32, 16)]
            v011 = val_v[pl.ds(e + 48, 16)]
            v100 = val_v[pl.ds(e + 64, 16)]
            v101 = val_v[pl.ds(e + 80, 16)]
            v110 = val_v[pl.ds(e + 96, 16)]
            v111 = val_v[pl.ds(e + 112, 16)]
            dx = frc_v[0, pl.ds(g * 16, 16)]
            dy = frc_v[1, pl.ds(g * 16, 16)]
            dz = frc_v[2, pl.ds(g * 16, 16)]
            v00 = v000 + dz * (v001 - v000)
            v01 = v010 + dz * (v011 - v010)
            v10 = v100 + dz * (v101 - v100)
            v11 = v110 + dz * (v111 - v110)
            v0 = v00 + dy * (v01 - v00)
            v1 = v10 + dy * (v11 - v10)
            row_v[pl.ds(g * 16, 16)] = v0 + dx * (v1 - v0)

    def fire_out(r, b):
        s = r // _H
        i = r % _H
        pltpu.make_async_copy(rows[b], out_hbm.at[s, i], osem[b]).start()

    def wait_out(r, b):
        s = r // _H
        i = r % _H
        pltpu.make_async_copy(rows[b], out_hbm.at[s, i], osem[b]).wait()

    compute_idx(wid, 0)
    fire(0)

    def body(u, carry):
        r0 = wid + 2 * u * _NW
        compute_idx(r0 + _NW, 1)
        fire(1)
        drain(0)
        pl.when(u > 0)(lambda: wait_out(r0 - 2 * _NW, 0))
        combine(0)
        fire_out(r0, 0)

        def prefetch():
            compute_idx(r0 + 2 * _NW, 0)
            fire(0)
        pl.when(u < _HALF - 1)(prefetch)
        drain(1)
        pl.when(u > 0)(lambda: wait_out(r0 - _NW, 1))
        combine(1)
        fire_out(r0 + _NW, 1)
        return carry

    lax.fori_loop(0, _HALF, body, 0)
    wait_out(wid + (_RPW - 2) * _NW, 0)
    wait_out(wid + (_RPW - 1) * _NW, 1)


_v2s_kernel = functools.partial(
    pl.kernel,
    out_type=jax.ShapeDtypeStruct((_S, _H, _W), jnp.float32),
    mesh=plsc.VectorSubcoreMesh(core_axis_name="core", subcore_axis_name="sub",
                                num_cores=_NC, num_subcores=_NS),
    scratch_types=_SCRATCH,
    compiler_params=pltpu.CompilerParams(needs_layout_passes=False),
)(_v2s_body)


@jax.jit
def kernel(vol, trf):
    v = vol.reshape(_N)
    t = trf.reshape(_S * 12)
    out3 = _v2s_kernel(v, t)
    return jnp.transpose(out3, (1, 2, 0))[None, ..., None]
